# baseline (speedup 1.0000x reference)
import functools

import jax
import jax.numpy as jnp
from jax import lax
from jax.experimental import pallas as pl
from jax.experimental.pallas import tpu as pltpu

_GH = _GW = 16
_HID = 32
_H0 = _W0 = 8
_R0 = _H0 * _W0
_H1 = _W1 = 4
_R1 = _H1 * _W1
_GB = 32
_PFN_CHUNKS = 32


def _mm(x, w, s, b, relu=True):
    y = jnp.dot(x, w, preferred_element_type=jnp.float32) * s + b
    return jnp.maximum(y, 0.0) if relu else y


def _tap_masks(rows, H, W):
    r = lax.broadcasted_iota(jnp.int32, (rows, 1), 0)
    hw = r % (H * W)
    h, w = hw // W, hw % W
    nl, nr, nt, nb = w != 0, w != W - 1, h != 0, h != H - 1
    tm = {(0, 0): nt & nl, (0, 1): nt, (0, 2): nt & nr,
          (1, 0): nl, (1, 2): nr,
          (2, 0): nb & nl, (2, 1): nb, (2, 2): nb & nr}
    return tm, nl, nt


def _patches_3x3(pad, base, W, rows, tm):
    taps = []
    for kh in range(3):
        for kw in range(3):
            t = pad[pl.ds(base + (kh - 1) * W + (kw - 1), rows), :]
            m = tm.get((kh, kw))
            if m is not None:
                t = jnp.where(m, t, 0.0)
            taps.append(t)
    return jnp.concatenate(taps, axis=-1)


def _conv3x3_s1(pad, base, W, rows, tm, w, s, b):
    return _mm(_patches_3x3(pad, base, W, rows, tm), w, s, b)


def _entry_s2(pad, base, W, rows, nl, nt, cin, w, s, b):
    r00 = pad[pl.ds(base, rows), :]
    rm1 = jnp.where(nl, pad[pl.ds(base - 1, rows), :], 0.0)
    ru0 = jnp.where(nt, pad[pl.ds(base - W, rows), :], 0.0)
    ru1 = jnp.where(nt & nl, pad[pl.ds(base - W - 1, rows), :], 0.0)
    blk = lambda v, i: v[:, i * cin:(i + 1) * cin]
    taps = [blk(ru1, 3), blk(ru0, 2), blk(ru0, 3),
            blk(rm1, 1), blk(r00, 0), blk(r00, 1),
            blk(rm1, 3), blk(r00, 2), blk(r00, 3)]
    return _mm(jnp.concatenate(taps, axis=-1), w, s, b)


def _s2d_rows(x, G, H, W, C):
    x = x.reshape(G, H // 2, 2, W // 2, 2, C)
    R = (H // 2) * (W // 2)
    return jnp.concatenate(
        [x[:, :, ph, :, pw, :].reshape(G * R, C) for ph in (0, 1) for pw in (0, 1)],
        axis=-1)


def _rpn_head_kernel(f_ref, id_ref,
                     b0_we, b0_se, b0_be, b0_wr, b0_sr, b0_br,
                     b1_we, b1_se, b1_be, b1_wr, b1_sr, b1_br,
                     d0_w, d0_s, d0_b, d1_w, d1_s, d1_b,
                     w_sh, s_sh, b_sh, w_mid, s_mid, b_mid, w_fin, b_fin,
                     o_ref,
                     pad0, pada, pad1i, pad1, padu, pads, padm,
                     *, G, n0, n1, npts):
    rows0, rows1 = G * _R0, G * _R1
    B0, B1 = 16, 8
    C0 = 4 * _HID
    tm0, nl0, nt0 = _tap_masks(rows0, _H0, _W0)
    tm1, nl1, nt1 = _tap_masks(rows1, _H1, _W1)

    pad0[pl.ds(B0, rows0), :] = jnp.zeros((rows0, C0), jnp.float32)
    cc_i = lax.broadcasted_iota(jnp.int32, (1, _R0, C0), 1)
    lb_i = lax.broadcasted_iota(jnp.int32, (1, _R0, C0), 2) // _HID
    ci = cc_i * 4 + lb_i
    PG = 16

    def _scatter_group(g8, carry):
        fg = f_ref[:, pl.ds(g8 * PG, PG), :]
        fg4 = jnp.concatenate([fg, fg, fg, fg], axis=-1)
        ig = id_ref[pl.ds(g8, 1), :, :].reshape(G, PG)
        sel = [jnp.where(ig[:, j:j + 1][:, :, None] == ci,
                         fg4[:, j:j + 1, :], 0.0)
               for j in range(PG)]
        while len(sel) > 1:
            sel = [jnp.maximum(sel[i], sel[i + 1]) for i in range(0, len(sel), 2)]
        accv = pad0[pl.ds(B0, rows0), :].reshape(G, _R0, C0)
        pad0[pl.ds(B0, rows0), :] = jnp.maximum(accv, sel[0]).reshape(rows0, C0)
        return carry

    lax.fori_loop(0, npts // PG, _scatter_group, 0)

    y = _entry_s2(pad0, B0, _W0, rows0, nl0, nt0, _HID,
                  b0_we[...], b0_se[...], b0_be[...])
    for i in range(n0):
        pada[pl.ds(B0, rows0), :] = y
        y = _conv3x3_s1(pada, B0, _W0, rows0, tm0, b0_wr[i], b0_sr[i], b0_br[i])

    c0 = y.shape[-1]
    s2d1 = _s2d_rows(y.reshape(G, _H0, _W0, c0), G, _H0, _W0, c0)
    pad1i[pl.ds(B1, rows1), :] = s2d1

    y = _entry_s2(pad1i, B1, _W1, rows1, nl1, nt1, c0,
                  b1_we[...], b1_se[...], b1_be[...])
    for i in range(n1):
        pad1[pl.ds(B1, rows1), :] = y
        y = _conv3x3_s1(pad1, B1, _W1, rows1, tm1, b1_wr[i], b1_sr[i], b1_br[i])

    u0 = _mm(s2d1, d0_w[...], d0_s[...], d0_b[...])
    u1 = _mm(y, d1_w[...], d1_s[...], d1_b[...])
    padu[pl.ds(B1, rows1), :] = jnp.concatenate([u0, u1], axis=-1)

    ysh = _conv3x3_s1(padu, B1, _W1, rows1, tm1, w_sh[...], s_sh[...], b_sh[...])
    pads[pl.ds(B1, rows1), :] = ysh
    ymid = _conv3x3_s1(pads, B1, _W1, rows1, tm1, w_mid[...], s_mid[...], b_mid[...])
    padm[pl.ds(B1, rows1), :] = ymid
    patches = _patches_3x3(padm, B1, _W1, rows1, tm1)
    o_ref[...] = jnp.dot(patches, w_fin[...],
                         preferred_element_type=jnp.float32) + b_fin[...]


def _rpn_head_kernel_sm(x_ref,
                        b0_we, b0_se, b0_be, b0_wr, b0_sr, b0_br,
                        b1_we, b1_se, b1_be, b1_wr, b1_sr, b1_br,
                        d0_w, d0_s, d0_b, d1_w, d1_s, d1_b,
                        w_sh, s_sh, b_sh, w_mid, s_mid, b_mid, w_fin, b_fin,
                        o_ref,
                        pad0, pada, pad1i, pad1, padu, pads, padm,
                        *, G, n0, n1):
    rows0, rows1 = G * _R0, G * _R1
    B0, B1 = 16, 8
    tm0, nl0, nt0 = _tap_masks(rows0, _H0, _W0)
    tm1, nl1, nt1 = _tap_masks(rows1, _H1, _W1)

    bev = jnp.maximum(x_ref[...], 0.0)
    s2d0 = _s2d_rows(bev.reshape(G, _GH, _GW, _HID), G, _GH, _GW, _HID)
    pad0[pl.ds(B0, rows0), :] = s2d0

    y = _entry_s2(pad0, B0, _W0, rows0, nl0, nt0, _HID,
                  b0_we[...], b0_se[...], b0_be[...])
    for i in range(n0):
        pada[pl.ds(B0, rows0), :] = y
        y = _conv3x3_s1(pada, B0, _W0, rows0, tm0, b0_wr[i], b0_sr[i], b0_br[i])

    c0 = y.shape[-1]
    s2d1 = _s2d_rows(y.reshape(G, _H0, _W0, c0), G, _H0, _W0, c0)
    pad1i[pl.ds(B1, rows1), :] = s2d1

    y = _entry_s2(pad1i, B1, _W1, rows1, nl1, nt1, c0,
                  b1_we[...], b1_se[...], b1_be[...])
    for i in range(n1):
        pad1[pl.ds(B1, rows1), :] = y
        y = _conv3x3_s1(pad1, B1, _W1, rows1, tm1, b1_wr[i], b1_sr[i], b1_br[i])

    u0 = _mm(s2d1, d0_w[...], d0_s[...], d0_b[...])
    u1 = _mm(y, d1_w[...], d1_s[...], d1_b[...])
    padu[pl.ds(B1, rows1), :] = jnp.concatenate([u0, u1], axis=-1)

    ysh = _conv3x3_s1(padu, B1, _W1, rows1, tm1, w_sh[...], s_sh[...], b_sh[...])
    pads[pl.ds(B1, rows1), :] = ysh
    ymid = _conv3x3_s1(pads, B1, _W1, rows1, tm1, w_mid[...], s_mid[...], b_mid[...])
    padm[pl.ds(B1, rows1), :] = ymid
    patches = _patches_3x3(padm, B1, _W1, rows1, tm1)
    o_ref[...] = jnp.dot(patches, w_fin[...],
                         preferred_element_type=jnp.float32) + b_fin[...]


def _const(a):
    z = (0,) * a.ndim
    return pl.BlockSpec(a.shape, lambda i, _z=z: _z)


def _wts_scratch(b0, b1, d0, d1, hd, G):
    rows0, rows1 = G * _R0, G * _R1
    c0 = b0["wr"].shape[-1]
    ds1 = b1["wr"].shape[-1]
    head_in = hd["w_sh"].shape[0] // 9
    mid_c = hd["w_mid"].shape[-1]
    weights = (b0["we"], b0["se"], b0["be"], b0["wr"], b0["sr"], b0["br"],
               b1["we"], b1["se"], b1["be"], b1["wr"], b1["sr"], b1["br"],
               d0["w"], d0["s"], d0["b"], d1["w"], d1["s"], d1["b"],
               hd["w_sh"], hd["s_sh"], hd["b_sh"],
               hd["w_mid"], hd["s_mid"], hd["b_mid"],
               hd["w_fin"], hd["b_fin"])
    scratch = [pltpu.VMEM((16 + rows0 + 16, 4 * _HID), jnp.float32),
               pltpu.VMEM((16 + rows0 + 16, c0), jnp.float32),
               pltpu.VMEM((8 + rows1 + 8, 4 * c0), jnp.float32),
               pltpu.VMEM((8 + rows1 + 8, ds1), jnp.float32),
               pltpu.VMEM((8 + rows1 + 8, head_in), jnp.float32),
               pltpu.VMEM((8 + rows1 + 8, ds1), jnp.float32),
               pltpu.VMEM((8 + rows1 + 8, mid_c), jnp.float32)]
    return weights, scratch


def _rpn_head_call(feats3, ids, b0, b1, d0, d1, hd, *, G):
    N, npts = feats3.shape[0], feats3.shape[1]
    rows1 = G * _R1
    tot = hd["w_fin"].shape[-1]
    kern = functools.partial(_rpn_head_kernel, G=G, npts=npts,
                             n0=b0["wr"].shape[0], n1=b1["wr"].shape[0])
    weights, scratch = _wts_scratch(b0, b1, d0, d1, hd, G)
    return pl.pallas_call(
        kern,
        out_shape=jax.ShapeDtypeStruct((N * _R1, tot), jnp.float32),
        grid=(N // G,),
        in_specs=[pl.BlockSpec((G, npts, _HID), lambda i: (i, 0, 0)),
                  pl.BlockSpec((npts // 16, G, 16), lambda i: (0, i, 0))]
                 + [_const(a) for a in weights],
        out_specs=pl.BlockSpec((rows1, tot), lambda i: (i, 0)),
        scratch_shapes=scratch,
        compiler_params=pltpu.CompilerParams(dimension_semantics=("parallel",)),
    )(feats3, ids, *weights)


def _rpn_head_call_sm(segmax3, b0, b1, d0, d1, hd, *, G):
    N = segmax3.shape[0]
    rows1 = G * _R1
    tot = hd["w_fin"].shape[-1]
    kern = functools.partial(_rpn_head_kernel_sm, G=G,
                             n0=b0["wr"].shape[0], n1=b1["wr"].shape[0])
    weights, scratch = _wts_scratch(b0, b1, d0, d1, hd, G)
    return pl.pallas_call(
        kern,
        out_shape=jax.ShapeDtypeStruct((N * _R1, tot), jnp.float32),
        grid=(N // G,),
        in_specs=[pl.BlockSpec((G, _GH * _GW, _HID), lambda i: (i, 0, 0))]
                 + [_const(a) for a in weights],
        out_specs=pl.BlockSpec((rows1, tot), lambda i: (i, 0)),
        scratch_shapes=scratch,
        compiler_params=pltpu.CompilerParams(dimension_semantics=("parallel",)),
    )(segmax3, *weights)


def _pfn_kernel(x_ref, w_ref, s_ref, b_ref, o_ref):
    o_ref[...] = jnp.maximum(
        jnp.dot(x_ref[...], w_ref[...], preferred_element_type=jnp.float32)
        * s_ref[...] + b_ref[...], 0.0)


def _pfn_call(x, w, s, b):
    rows, hid = x.shape[0], w.shape[1]
    chunks = _PFN_CHUNKS if rows % (_PFN_CHUNKS * 8) == 0 else 1
    m = rows // chunks
    return pl.pallas_call(
        _pfn_kernel,
        out_shape=jax.ShapeDtypeStruct((rows, hid), jnp.float32),
        grid=(chunks,),
        in_specs=[pl.BlockSpec((m, x.shape[1]), lambda i: (i, 0)),
                  _const(w), _const(s), _const(b)],
        out_specs=pl.BlockSpec((m, hid), lambda i: (i, 0)),
        compiler_params=pltpu.CompilerParams(dimension_semantics=("parallel",)),
    )(x, w, s, b)


def kernel(pcls, img_t0,
           pfn_wT, pfn_scale, pfn_bias,
           b0_we, b0_se, b0_be, b0_wr, b0_sr, b0_br,
           b1_we, b1_se, b1_be, b1_wr, b1_sr, b1_br,
           d0_w, d0_s, d0_b,
           d1_w, d1_s, d1_b,
           h_w_sh, h_s_sh, h_b_sh,
           h_w_mid, h_s_mid, h_b_mid,
           h_w_finT, h_b_finT):
    del img_t0
    B_, NP, F_ = pcls.shape

    feats = _pfn_call(pcls.reshape(B_ * NP, F_),
                      jnp.transpose(pfn_wT),
                      pfn_scale.reshape(1, -1), pfn_bias.reshape(1, -1))

    px = jnp.clip(jnp.floor(pcls[..., 0]), 0, _GW - 1).astype(jnp.int32)
    py = jnp.clip(jnp.floor(pcls[..., 1]), 0, _GH - 1).astype(jnp.int32)
    cell = py * _GW + px
    ids = ((py // 2) * (_GW // 2) + px // 2) * 4 + (py % 2) * 2 + (px % 2)
    ids = jnp.transpose(ids.reshape(B_, NP // 16, 16), (1, 0, 2))

    G = _GB
    while B_ % G:
        G //= 2
    hd = {"w_sh": h_w_sh, "s_sh": h_s_sh, "b_sh": h_b_sh,
          "w_mid": h_w_mid, "s_mid": h_s_mid, "b_mid": h_b_mid,
          "w_fin": jnp.transpose(h_w_finT), "b_fin": jnp.transpose(h_b_finT)}
    b0 = {"we": b0_we, "se": b0_se, "be": b0_be,
          "wr": b0_wr, "sr": b0_sr, "br": b0_br}
    b1 = {"we": b1_we, "se": b1_se, "be": b1_be,
          "wr": b1_wr, "sr": b1_sr, "br": b1_br}
    d0 = {"w": d0_w, "s": d0_s, "b": d0_b}
    d1 = {"w": d1_w, "s": d1_s, "b": d1_b}

    Bt = min(B_, max(G, (13 * B_ // 24) // G * G))
    Bs = B_ - Bt
    feats3 = feats.reshape(B_, NP, _HID)
    outs = [_rpn_head_call(feats3[:Bt], ids[:, :Bt], b0, b1, d0, d1, hd, G=G)]
    if Bs:
        local = jnp.arange(Bs, dtype=jnp.int32)[:, None] * (_GH * _GW)
        flat = (local + cell[Bt:]).reshape(-1)
        seg_max = jax.ops.segment_max(feats3[Bt:].reshape(Bs * NP, _HID),
                                      flat, num_segments=Bs * _GH * _GW)
        outs.append(_rpn_head_call_sm(seg_max.reshape(Bs, _GH * _GW, _HID),
                                      b0, b1, d0, d1, hd, G=G))
    out = jnp.concatenate(outs, axis=0) if Bs else outs[0]
    out = out.reshape(B_, _H1, _W1, out.shape[-1])

    occ = jnp.max((cell[:, :, None] ==
                   jnp.arange(_GH * _GW, dtype=jnp.int32)[None, None, :])
                  .astype(jnp.float32), axis=1)
    occ = occ.reshape(B_, _GH, _GW)[:, None, :, :]

    pred = {"pos": out[..., 0:2], "z": out[..., 2:3], "dims": out[..., 3:6],
            "rot": out[..., 6:8], "probs": out[..., 8:9]}
    return pred, {"bev_net_input_dbg": occ}


# speedup vs baseline: 1.1940x; 1.1940x over previous
import functools

import jax
import jax.numpy as jnp
from jax import lax
from jax.experimental import pallas as pl
from jax.experimental.pallas import tpu as pltpu

_GH = _GW = 16
_HID = 32
_H0 = _W0 = 8
_R0 = _H0 * _W0
_H1 = _W1 = 4
_R1 = _H1 * _W1
_GB = 32
_PFN_CHUNKS = 32


def _mm(x, w, s, b, relu=True):
    y = jnp.dot(x, w, preferred_element_type=jnp.float32) * s + b
    return jnp.maximum(y, 0.0) if relu else y


def _tap_masks(rows, H, W):
    r = lax.broadcasted_iota(jnp.int32, (rows, 1), 0)
    hw = r % (H * W)
    h, w = hw // W, hw % W
    nl, nr, nt, nb = w != 0, w != W - 1, h != 0, h != H - 1
    tm = {(0, 0): nt & nl, (0, 1): nt, (0, 2): nt & nr,
          (1, 0): nl, (1, 2): nr,
          (2, 0): nb & nl, (2, 1): nb, (2, 2): nb & nr}
    return tm, nl, nt


def _patches_3x3(pad, base, W, rows, tm):
    taps = []
    for kh in range(3):
        for kw in range(3):
            t = pad[pl.ds(base + (kh - 1) * W + (kw - 1), rows), :]
            m = tm.get((kh, kw))
            if m is not None:
                t = jnp.where(m, t, 0.0)
            taps.append(t)
    return jnp.concatenate(taps, axis=-1)


def _conv3x3_s1(pad, base, W, rows, tm, w, s, b):
    return _mm(_patches_3x3(pad, base, W, rows, tm), w, s, b)


def _entry_s2(pad, base, W, rows, nl, nt, cin, w, s, b):
    r00 = pad[pl.ds(base, rows), :]
    rm1 = jnp.where(nl, pad[pl.ds(base - 1, rows), :], 0.0)
    ru0 = jnp.where(nt, pad[pl.ds(base - W, rows), :], 0.0)
    ru1 = jnp.where(nt & nl, pad[pl.ds(base - W - 1, rows), :], 0.0)
    blk = lambda v, i: v[:, i * cin:(i + 1) * cin]
    taps = [blk(ru1, 3), blk(ru0, 2), blk(ru0, 3),
            blk(rm1, 1), blk(r00, 0), blk(r00, 1),
            blk(rm1, 3), blk(r00, 2), blk(r00, 3)]
    return _mm(jnp.concatenate(taps, axis=-1), w, s, b)


def _s2d_rows(x, G, H, W, C):
    x = x.reshape(G, H // 2, 2, W // 2, 2, C)
    R = (H // 2) * (W // 2)
    return jnp.concatenate(
        [x[:, :, ph, :, pw, :].reshape(G * R, C) for ph in (0, 1) for pw in (0, 1)],
        axis=-1)


def _rpn_head_kernel(f_ref, id_ref,
                     b0_we, b0_se, b0_be, b0_wr, b0_sr, b0_br,
                     b1_we, b1_se, b1_be, b1_wr, b1_sr, b1_br,
                     d0_w, d0_s, d0_b, d1_w, d1_s, d1_b,
                     w_sh, s_sh, b_sh, w_mid, s_mid, b_mid, w_fin, b_fin,
                     o_ref,
                     pad0, pada, pad1i, pad1, padu, pads, padm,
                     *, G, n0, n1, npts):
    rows0, rows1 = G * _R0, G * _R1
    B0, B1 = 16, 8
    C0 = 4 * _HID
    tm0, nl0, nt0 = _tap_masks(rows0, _H0, _W0)
    tm1, nl1, nt1 = _tap_masks(rows1, _H1, _W1)

    pad0[pl.ds(B0, rows0), :] = jnp.zeros((rows0, C0), jnp.float32)
    cc_i = lax.broadcasted_iota(jnp.int32, (1, _R0, C0), 1)
    lb_i = lax.broadcasted_iota(jnp.int32, (1, _R0, C0), 2) // _HID
    ci = cc_i * 4 + lb_i
    PG = 16

    def _scatter_group(g8, carry):
        fg = f_ref[:, pl.ds(g8 * PG, PG), :]
        fg4 = jnp.concatenate([fg, fg, fg, fg], axis=-1)
        ig = id_ref[pl.ds(g8, 1), :, :].reshape(G, PG)
        sel = [jnp.where(ig[:, j:j + 1][:, :, None] == ci,
                         fg4[:, j:j + 1, :], 0.0)
               for j in range(PG)]
        while len(sel) > 1:
            sel = [jnp.maximum(sel[i], sel[i + 1]) for i in range(0, len(sel), 2)]
        accv = pad0[pl.ds(B0, rows0), :].reshape(G, _R0, C0)
        pad0[pl.ds(B0, rows0), :] = jnp.maximum(accv, sel[0]).reshape(rows0, C0)
        return carry

    lax.fori_loop(0, npts // PG, _scatter_group, 0)

    y = _entry_s2(pad0, B0, _W0, rows0, nl0, nt0, _HID,
                  b0_we[...], b0_se[...], b0_be[...])
    for i in range(n0):
        pada[pl.ds(B0, rows0), :] = y
        y = _conv3x3_s1(pada, B0, _W0, rows0, tm0, b0_wr[i], b0_sr[i], b0_br[i])

    c0 = y.shape[-1]
    s2d1 = _s2d_rows(y.reshape(G, _H0, _W0, c0), G, _H0, _W0, c0)
    pad1i[pl.ds(B1, rows1), :] = s2d1

    y = _entry_s2(pad1i, B1, _W1, rows1, nl1, nt1, c0,
                  b1_we[...], b1_se[...], b1_be[...])
    for i in range(n1):
        pad1[pl.ds(B1, rows1), :] = y
        y = _conv3x3_s1(pad1, B1, _W1, rows1, tm1, b1_wr[i], b1_sr[i], b1_br[i])

    u0 = _mm(s2d1, d0_w[...], d0_s[...], d0_b[...])
    u1 = _mm(y, d1_w[...], d1_s[...], d1_b[...])
    padu[pl.ds(B1, rows1), :] = jnp.concatenate([u0, u1], axis=-1)

    ysh = _conv3x3_s1(padu, B1, _W1, rows1, tm1, w_sh[...], s_sh[...], b_sh[...])
    pads[pl.ds(B1, rows1), :] = ysh
    ymid = _conv3x3_s1(pads, B1, _W1, rows1, tm1, w_mid[...], s_mid[...], b_mid[...])
    padm[pl.ds(B1, rows1), :] = ymid
    patches = _patches_3x3(padm, B1, _W1, rows1, tm1)
    o_ref[...] = jnp.dot(patches, w_fin[...],
                         preferred_element_type=jnp.float32) + b_fin[...]


def _rpn_head_kernel_sm(x_ref,
                        b0_we, b0_se, b0_be, b0_wr, b0_sr, b0_br,
                        b1_we, b1_se, b1_be, b1_wr, b1_sr, b1_br,
                        d0_w, d0_s, d0_b, d1_w, d1_s, d1_b,
                        w_sh, s_sh, b_sh, w_mid, s_mid, b_mid, w_fin, b_fin,
                        o_ref,
                        pad0, pada, pad1i, pad1, padu, pads, padm,
                        *, G, n0, n1):
    rows0, rows1 = G * _R0, G * _R1
    B0, B1 = 16, 8
    tm0, nl0, nt0 = _tap_masks(rows0, _H0, _W0)
    tm1, nl1, nt1 = _tap_masks(rows1, _H1, _W1)

    bev = jnp.maximum(x_ref[...], 0.0)
    s2d0 = _s2d_rows(bev.reshape(G, _GH, _GW, _HID), G, _GH, _GW, _HID)
    pad0[pl.ds(B0, rows0), :] = s2d0

    y = _entry_s2(pad0, B0, _W0, rows0, nl0, nt0, _HID,
                  b0_we[...], b0_se[...], b0_be[...])
    for i in range(n0):
        pada[pl.ds(B0, rows0), :] = y
        y = _conv3x3_s1(pada, B0, _W0, rows0, tm0, b0_wr[i], b0_sr[i], b0_br[i])

    c0 = y.shape[-1]
    s2d1 = _s2d_rows(y.reshape(G, _H0, _W0, c0), G, _H0, _W0, c0)
    pad1i[pl.ds(B1, rows1), :] = s2d1

    y = _entry_s2(pad1i, B1, _W1, rows1, nl1, nt1, c0,
                  b1_we[...], b1_se[...], b1_be[...])
    for i in range(n1):
        pad1[pl.ds(B1, rows1), :] = y
        y = _conv3x3_s1(pad1, B1, _W1, rows1, tm1, b1_wr[i], b1_sr[i], b1_br[i])

    u0 = _mm(s2d1, d0_w[...], d0_s[...], d0_b[...])
    u1 = _mm(y, d1_w[...], d1_s[...], d1_b[...])
    padu[pl.ds(B1, rows1), :] = jnp.concatenate([u0, u1], axis=-1)

    ysh = _conv3x3_s1(padu, B1, _W1, rows1, tm1, w_sh[...], s_sh[...], b_sh[...])
    pads[pl.ds(B1, rows1), :] = ysh
    ymid = _conv3x3_s1(pads, B1, _W1, rows1, tm1, w_mid[...], s_mid[...], b_mid[...])
    padm[pl.ds(B1, rows1), :] = ymid
    patches = _patches_3x3(padm, B1, _W1, rows1, tm1)
    o_ref[...] = jnp.dot(patches, w_fin[...],
                         preferred_element_type=jnp.float32) + b_fin[...]


def _const(a):
    z = (0,) * a.ndim
    return pl.BlockSpec(a.shape, lambda i, _z=z: _z)


def _wts_scratch(b0, b1, d0, d1, hd, G):
    rows0, rows1 = G * _R0, G * _R1
    c0 = b0["wr"].shape[-1]
    ds1 = b1["wr"].shape[-1]
    head_in = hd["w_sh"].shape[0] // 9
    mid_c = hd["w_mid"].shape[-1]
    weights = (b0["we"], b0["se"], b0["be"], b0["wr"], b0["sr"], b0["br"],
               b1["we"], b1["se"], b1["be"], b1["wr"], b1["sr"], b1["br"],
               d0["w"], d0["s"], d0["b"], d1["w"], d1["s"], d1["b"],
               hd["w_sh"], hd["s_sh"], hd["b_sh"],
               hd["w_mid"], hd["s_mid"], hd["b_mid"],
               hd["w_fin"], hd["b_fin"])
    scratch = [pltpu.VMEM((16 + rows0 + 16, 4 * _HID), jnp.float32),
               pltpu.VMEM((16 + rows0 + 16, c0), jnp.float32),
               pltpu.VMEM((8 + rows1 + 8, 4 * c0), jnp.float32),
               pltpu.VMEM((8 + rows1 + 8, ds1), jnp.float32),
               pltpu.VMEM((8 + rows1 + 8, head_in), jnp.float32),
               pltpu.VMEM((8 + rows1 + 8, ds1), jnp.float32),
               pltpu.VMEM((8 + rows1 + 8, mid_c), jnp.float32)]
    return weights, scratch


def _rpn_head_call(feats3, ids, b0, b1, d0, d1, hd, *, G):
    N, npts = feats3.shape[0], feats3.shape[1]
    rows1 = G * _R1
    tot = hd["w_fin"].shape[-1]
    kern = functools.partial(_rpn_head_kernel, G=G, npts=npts,
                             n0=b0["wr"].shape[0], n1=b1["wr"].shape[0])
    weights, scratch = _wts_scratch(b0, b1, d0, d1, hd, G)
    return pl.pallas_call(
        kern,
        out_shape=jax.ShapeDtypeStruct((N * _R1, tot), jnp.float32),
        grid=(N // G,),
        in_specs=[pl.BlockSpec((G, npts, _HID), lambda i: (i, 0, 0)),
                  pl.BlockSpec((npts // 16, G, 16), lambda i: (0, i, 0))]
                 + [_const(a) for a in weights],
        out_specs=pl.BlockSpec((rows1, tot), lambda i: (i, 0)),
        scratch_shapes=scratch,
        compiler_params=pltpu.CompilerParams(dimension_semantics=("parallel",)),
    )(feats3, ids, *weights)


def _rpn_head_call_sm(segmax3, b0, b1, d0, d1, hd, *, G):
    N = segmax3.shape[0]
    rows1 = G * _R1
    tot = hd["w_fin"].shape[-1]
    kern = functools.partial(_rpn_head_kernel_sm, G=G,
                             n0=b0["wr"].shape[0], n1=b1["wr"].shape[0])
    weights, scratch = _wts_scratch(b0, b1, d0, d1, hd, G)
    return pl.pallas_call(
        kern,
        out_shape=jax.ShapeDtypeStruct((N * _R1, tot), jnp.float32),
        grid=(N // G,),
        in_specs=[pl.BlockSpec((G, _GH * _GW, _HID), lambda i: (i, 0, 0))]
                 + [_const(a) for a in weights],
        out_specs=pl.BlockSpec((rows1, tot), lambda i: (i, 0)),
        scratch_shapes=scratch,
        compiler_params=pltpu.CompilerParams(dimension_semantics=("parallel",)),
    )(segmax3, *weights)


def _pfn_kernel(x_ref, w_ref, s_ref, b_ref, o_ref):
    o_ref[...] = jnp.maximum(
        jnp.dot(x_ref[...], w_ref[...], preferred_element_type=jnp.float32)
        * s_ref[...] + b_ref[...], 0.0)


def _pfn_call(x, w, s, b):
    rows, hid = x.shape[0], w.shape[1]
    chunks = _PFN_CHUNKS if rows % (_PFN_CHUNKS * 8) == 0 else 1
    m = rows // chunks
    return pl.pallas_call(
        _pfn_kernel,
        out_shape=jax.ShapeDtypeStruct((rows, hid), jnp.float32),
        grid=(chunks,),
        in_specs=[pl.BlockSpec((m, x.shape[1]), lambda i: (i, 0)),
                  _const(w), _const(s), _const(b)],
        out_specs=pl.BlockSpec((m, hid), lambda i: (i, 0)),
        compiler_params=pltpu.CompilerParams(dimension_semantics=("parallel",)),
    )(x, w, s, b)


def kernel(pcls, img_t0,
           pfn_wT, pfn_scale, pfn_bias,
           b0_we, b0_se, b0_be, b0_wr, b0_sr, b0_br,
           b1_we, b1_se, b1_be, b1_wr, b1_sr, b1_br,
           d0_w, d0_s, d0_b,
           d1_w, d1_s, d1_b,
           h_w_sh, h_s_sh, h_b_sh,
           h_w_mid, h_s_mid, h_b_mid,
           h_w_finT, h_b_finT):
    del img_t0
    B_, NP, F_ = pcls.shape

    px = jnp.clip(jnp.floor(pcls[..., 0]), 0, _GW - 1).astype(jnp.int32)
    py = jnp.clip(jnp.floor(pcls[..., 1]), 0, _GH - 1).astype(jnp.int32)
    cell = py * _GW + px
    ids = ((py // 2) * (_GW // 2) + px // 2) * 4 + (py % 2) * 2 + (px % 2)
    ids = jnp.transpose(ids.reshape(B_, NP // 16, 16), (1, 0, 2))

    G = _GB
    while B_ % G:
        G //= 2
    hd = {"w_sh": h_w_sh, "s_sh": h_s_sh, "b_sh": h_b_sh,
          "w_mid": h_w_mid, "s_mid": h_s_mid, "b_mid": h_b_mid,
          "w_fin": jnp.transpose(h_w_finT), "b_fin": jnp.transpose(h_b_finT)}
    b0 = {"we": b0_we, "se": b0_se, "be": b0_be,
          "wr": b0_wr, "sr": b0_sr, "br": b0_br}
    b1 = {"we": b1_we, "se": b1_se, "be": b1_be,
          "wr": b1_wr, "sr": b1_sr, "br": b1_br}
    d0 = {"w": d0_w, "s": d0_s, "b": d0_b}
    d1 = {"w": d1_w, "s": d1_s, "b": d1_b}

    Bt = min(B_, max(G, (2 * B_ // 5) // G * G))
    Bs = B_ - Bt
    wp, sp, bp = (jnp.transpose(pfn_wT), pfn_scale.reshape(1, -1),
                  pfn_bias.reshape(1, -1))
    feats_t = _pfn_call(pcls[:Bt].reshape(Bt * NP, F_), wp, sp, bp)
    outs = [_rpn_head_call(feats_t.reshape(Bt, NP, _HID), ids[:, :Bt],
                           b0, b1, d0, d1, hd, G=G)]
    if Bs:
        feats_s = _pfn_call(pcls[Bt:].reshape(Bs * NP, F_), wp, sp, bp)
        local = jnp.arange(Bs, dtype=jnp.int32)[:, None] * (_GH * _GW)
        flat = (local + cell[Bt:]).reshape(-1)
        seg_max = jax.ops.segment_max(feats_s, flat,
                                      num_segments=Bs * _GH * _GW)
        outs.append(_rpn_head_call_sm(seg_max.reshape(Bs, _GH * _GW, _HID),
                                      b0, b1, d0, d1, hd, G=G))
    out = jnp.concatenate(outs, axis=0) if Bs else outs[0]
    out = out.reshape(B_, _H1, _W1, out.shape[-1])

    occ = jnp.max((cell[:, :, None] ==
                   jnp.arange(_GH * _GW, dtype=jnp.int32)[None, None, :])
                  .astype(jnp.float32), axis=1)
    occ = occ.reshape(B_, _GH, _GW)[:, None, :, :]

    pred = {"pos": out[..., 0:2], "z": out[..., 2:3], "dims": out[..., 3:6],
            "rot": out[..., 6:8], "probs": out[..., 8:9]}
    return pred, {"bev_net_input_dbg": occ}
